# Initial kernel scaffold; baseline (speedup 1.0000x reference)
import jax
import jax.numpy as jnp
from jax.experimental import pallas as pl


def kernel(output, labels):
    raise NotImplementedError("write your pallas kernel here")


# speedup vs baseline: 2.7448x; 2.7448x over previous
import functools

import jax
import jax.numpy as jnp
from jax.experimental import pallas as pl

_NUM_HARD = 2
_LANES = 128


def _scan_kernel(o_ref, l_ref, stats_ref, cand_ref, *, k):
    i = pl.program_id(0)
    o = o_ref[...]
    l = l_ref[...]
    rsub = o.shape[1]

    cls = l[0]
    s0 = o[0]
    pos = cls > 0.5
    neg = cls < -0.5

    parts = [jnp.sum(jnp.where(pos, 1.0, 0.0), axis=0, keepdims=True)]
    for c in range(1, 5):
        d = o[c] - l[c]
        ad = jnp.abs(d)
        e = jnp.where(ad < 1.0, 0.5 * d * d, ad - 0.5)
        parts.append(jnp.sum(jnp.where(pos, e, 0.0), axis=0, keepdims=True))

    eps = 1e-12
    p = jnp.clip(jax.nn.sigmoid(s0), eps, 1.0 - eps)
    bce = -(cls * jnp.log(p) + (1.0 - cls) * jnp.log(1.0 - p))
    parts.append(jnp.sum(jnp.where(pos, bce, 0.0), axis=0, keepdims=True))
    parts.append(
        jnp.sum(jnp.where(pos & (p >= 0.5), 1.0, 0.0), axis=0, keepdims=True)
    )
    parts.append(jnp.zeros((1, _LANES), jnp.float32))
    pstack = jnp.concatenate(parts, axis=0)

    @pl.when(i == 0)
    def _():
        stats_ref[...] = pstack

    @pl.when(i > 0)
    def _():
        stats_ref[...] = stats_ref[...] + pstack

    scores = jnp.where(neg, s0, -jnp.inf)
    ridx = jax.lax.broadcasted_iota(jnp.int32, (rsub, _LANES), 0)
    lidx = jax.lax.broadcasted_iota(jnp.int32, (rsub, _LANES), 1)
    idx2 = ridx * _LANES + lidx
    lane_k = jax.lax.broadcasted_iota(jnp.int32, (1, _LANES), 1)

    def body(j, carry):
        sc, cand = carry
        m = jnp.max(sc)
        sel = jnp.min(jnp.where(sc == m, idx2, jnp.int32(2**30)))
        sc = jnp.where(idx2 == sel, -jnp.inf, sc)
        cand = jnp.where(lane_k == j, m, cand)
        return sc, cand

    _, cand = jax.lax.fori_loop(
        0, k, body, (scores, jnp.full((1, _LANES), -jnp.inf, jnp.float32))
    )
    cand_ref[...] = jnp.concatenate(
        [cand, jnp.full((7, _LANES), -jnp.inf, jnp.float32)], axis=0
    )


def _final_kernel(stats_ref, cand_ref, out_ref, *, k):
    stats = stats_ref[...]
    cand = cand_ref[...]
    t = cand.shape[0]

    pos_count = jnp.sum(stats[0:1, :])
    sl = [jnp.sum(stats[c : c + 1, :]) for c in range(1, 5)]
    bce_pos = jnp.sum(stats[5:6, :])
    pos_correct = jnp.sum(stats[6:7, :])

    ridx = jax.lax.broadcasted_iota(jnp.int32, (t, _LANES), 0)
    lidx = jax.lax.broadcasted_iota(jnp.int32, (t, _LANES), 1)
    idx2 = ridx * _LANES + lidx
    eps = 1e-12

    def body(j, carry):
        sc, bce_acc, negc_acc = carry
        m = jnp.max(sc)
        sel = jnp.min(jnp.where(sc == m, idx2, jnp.int32(2**30)))
        sc = jnp.where(idx2 == sel, -jnp.inf, sc)
        p = jnp.clip(jax.nn.sigmoid(m), eps, 1.0 - eps)
        bce_acc = bce_acc - jnp.log(1.0 - p)
        negc_acc = negc_acc + jnp.where(p < 0.5, 1.0, 0.0)
        return sc, bce_acc, negc_acc

    _, bce_neg, neg_correct = jax.lax.fori_loop(
        0, k, body, (cand, jnp.float32(0.0), jnp.float32(0.0))
    )

    classify = 0.5 * bce_pos / pos_count + 0.5 * bce_neg / jnp.float32(k)
    rl = [s / pos_count for s in sl]
    loss = classify + rl[0] + rl[1] + rl[2] + rl[3]

    vals = [loss, classify, rl[0], rl[1], rl[2], rl[3], pos_correct,
            pos_count, neg_correct]
    lane = jax.lax.broadcasted_iota(jnp.int32, (1, _LANES), 1)
    out = jnp.zeros((1, _LANES), jnp.float32)
    for idx, v in enumerate(vals):
        out = jnp.where(lane == idx, v, out)
    out_ref[...] = out


@jax.jit
def kernel(output, labels):
    batch = labels.shape[0]
    k = _NUM_HARD * batch
    r = output.shape[0] * output.shape[1]
    rows128 = r // _LANES

    o3 = output.reshape(-1, 5).T.reshape(5, rows128, _LANES)
    l3 = labels.reshape(-1, 5).T.reshape(5, rows128, _LANES)

    tile_rsub = 512
    num_tiles = rows128 // tile_rsub

    stats, cand = pl.pallas_call(
        functools.partial(_scan_kernel, k=k),
        grid=(num_tiles,),
        in_specs=[
            pl.BlockSpec((5, tile_rsub, _LANES), lambda i: (0, i, 0)),
            pl.BlockSpec((5, tile_rsub, _LANES), lambda i: (0, i, 0)),
        ],
        out_specs=[
            pl.BlockSpec((8, _LANES), lambda i: (0, 0)),
            pl.BlockSpec((8, _LANES), lambda i: (i, 0)),
        ],
        out_shape=[
            jax.ShapeDtypeStruct((8, _LANES), jnp.float32),
            jax.ShapeDtypeStruct((num_tiles * 8, _LANES), jnp.float32),
        ],
    )(o3, l3)

    res = pl.pallas_call(
        functools.partial(_final_kernel, k=k),
        out_shape=jax.ShapeDtypeStruct((1, _LANES), jnp.float32),
    )(stats, cand)[0]

    return (
        res[0],
        res[1],
        res[2],
        res[3],
        res[4],
        res[5],
        res[6].astype(jnp.int32),
        res[7].astype(jnp.int32),
        res[8].astype(jnp.int32),
        jnp.asarray(k, dtype=jnp.int32),
    )
